# Initial kernel scaffold; baseline (speedup 1.0000x reference)
import jax
import jax.numpy as jnp
from jax.experimental import pallas as pl


def kernel(c, img, q, cap_len, table):
    raise NotImplementedError("write your pallas kernel here")


# speedup vs baseline: 4.1997x; 4.1997x over previous
import functools

import jax
import jax.numpy as jnp
from jax import lax
from jax.experimental import pallas as pl
from jax.experimental.pallas import tpu as pltpu
from jax.experimental.pallas import tpu_sc as plsc

EMBED_DIM = 64
NC = 2
NS = 16
NW = NC * NS
CHUNK = 128


@functools.lru_cache(maxsize=None)
def _build_gather(total_rows):
    rows_per_w = total_rows // NW
    n_chunks = rows_per_w // CHUNK
    mesh = plsc.VectorSubcoreMesh(core_axis_name="c", subcore_axis_name="s")

    @functools.partial(
        pl.kernel,
        out_type=jax.ShapeDtypeStruct((total_rows, EMBED_DIM), jnp.float32),
        mesh=mesh,
        compiler_params=pltpu.CompilerParams(use_tc_tiling_on_sc=False),
        scratch_types=[
            pltpu.VMEM((n_chunks, CHUNK), jnp.int32),
            pltpu.VMEM((CHUNK, EMBED_DIM), jnp.float32),
            pltpu.VMEM((CHUNK, EMBED_DIM), jnp.float32),
            pltpu.SemaphoreType.DMA,
            pltpu.SemaphoreType.DMA,
            pltpu.SemaphoreType.DMA,
            pltpu.SemaphoreType.DMA,
        ],
    )
    def gather_kernel(idx_hbm, table_hbm, out_hbm, idx_v, buf0, buf1,
                      g0, g1, s0, s1):
        wid = lax.axis_index("s") * NC + lax.axis_index("c")
        base = wid * rows_per_w
        pltpu.sync_copy(idx_hbm.at[wid], idx_v)
        bufs = (buf0, buf1)
        gsems = (g0, g1)
        ssems = (s0, s1)
        gathers = [None, None]
        scatters = [None, None]
        gathers[0] = pltpu.async_copy(table_hbm.at[idx_v.at[0]], buf0, g0)
        for i in range(n_chunks):
            cur = i % 2
            nxt = (i + 1) % 2
            if i + 1 < n_chunks:
                if scatters[nxt] is not None:
                    scatters[nxt].wait()
                gathers[nxt] = pltpu.async_copy(
                    table_hbm.at[idx_v.at[i + 1]], bufs[nxt], gsems[nxt])
            gathers[cur].wait()
            scatters[cur] = pltpu.async_copy(
                bufs[cur], out_hbm.at[pl.ds(base + i * CHUNK, CHUNK)],
                ssems[cur])
        if n_chunks >= 2:
            scatters[(n_chunks - 2) % 2].wait()
        scatters[(n_chunks - 1) % 2].wait()

    return gather_kernel


def kernel(c, img, q, cap_len, table):
    batch, cap_len_dim = c.shape
    total_rows = batch * cap_len_dim
    idx = c.reshape(NW, total_rows // NW // CHUNK, CHUNK).astype(jnp.int32)
    flat = _build_gather(total_rows)(idx, table)
    c_emb = flat.reshape(batch, cap_len_dim, EMBED_DIM)
    return (img, c_emb, c, cap_len)


# speedup vs baseline: 4.3040x; 1.0248x over previous
import functools

import jax
import jax.numpy as jnp
from jax import lax
from jax.experimental import pallas as pl
from jax.experimental.pallas import tpu as pltpu
from jax.experimental.pallas import tpu_sc as plsc

EMBED_DIM = 64
NC = 2
NS = 16
NW = NC * NS
CHUNK = 128


NBUF = 8


@functools.lru_cache(maxsize=None)
def _build_gather(total_rows):
    rows_per_w = total_rows // NW
    n_chunks = rows_per_w // CHUNK
    mesh = plsc.VectorSubcoreMesh(core_axis_name="c", subcore_axis_name="s")

    @functools.partial(
        pl.kernel,
        out_type=jax.ShapeDtypeStruct((total_rows, EMBED_DIM), jnp.float32),
        mesh=mesh,
        compiler_params=pltpu.CompilerParams(use_tc_tiling_on_sc=False),
        scratch_types=[
            pltpu.VMEM((n_chunks, CHUNK), jnp.int32),
        ] + [pltpu.VMEM((CHUNK, EMBED_DIM), jnp.float32)] * NBUF
          + [pltpu.SemaphoreType.DMA] * (2 * NBUF),
    )
    def gather_kernel(idx_hbm, table_hbm, out_hbm, idx_v, *rest):
        bufs = rest[:NBUF]
        gsems = rest[NBUF:2 * NBUF]
        ssems = rest[2 * NBUF:]
        wid = lax.axis_index("s") * NC + lax.axis_index("c")
        base = wid * rows_per_w
        pltpu.sync_copy(idx_hbm.at[wid], idx_v)
        gathers = [None] * NBUF
        scatters = [None] * NBUF
        for i in range(min(NBUF - 1, n_chunks)):
            gathers[i] = pltpu.async_copy(
                table_hbm.at[idx_v.at[i]], bufs[i], gsems[i])
        for i in range(n_chunks):
            b = i % NBUF
            j = i + NBUF - 1
            if j < n_chunks:
                jb = j % NBUF
                if scatters[jb] is not None:
                    scatters[jb].wait()
                    scatters[jb] = None
                gathers[jb] = pltpu.async_copy(
                    table_hbm.at[idx_v.at[j]], bufs[jb], gsems[jb])
            gathers[b].wait()
            scatters[b] = pltpu.async_copy(
                bufs[b], out_hbm.at[pl.ds(base + i * CHUNK, CHUNK)],
                ssems[b])
        for s in scatters:
            if s is not None:
                s.wait()

    return gather_kernel


def kernel(c, img, q, cap_len, table):
    batch, cap_len_dim = c.shape
    total_rows = batch * cap_len_dim
    idx = c.reshape(NW, total_rows // NW // CHUNK, CHUNK).astype(jnp.int32)
    flat = _build_gather(total_rows)(idx, table)
    c_emb = flat.reshape(batch, cap_len_dim, EMBED_DIM)
    return (img, c_emb, c, cap_len)
